# baseline (speedup 1.0000x reference)
import jax
import jax.numpy as jnp
from jax.experimental import pallas as pl
from jax.experimental.pallas import tpu as pltpu

R_MIN = 0.5
R_MAX = 30.0
FLS_H = 512
FLS_W = 512
FOV_H = 130.0
BUFF = 16
P = 256
PS = 8
T = 8
C = 64
DOWN = 4
FH = FLS_H // DOWN
FW = FLS_W // DOWN
SLOT = 2 * P * T
MAX_EDGES = BUFF * SLOT
NPIX = FLS_H * FLS_W
NBLK = 256
KCAND = 512


def _body(xbT_ref, sc3_ref, wcat_ref, fn_ref, ts_ref, time_ref, poses_ref,
          pstate_ref, sframe_ref, i_ref, j_ref, w_ref, v_ref,
          fmap_hbm_in, imap_hbm_in,
          fmap_hbm_out, imap_hbm_out, fmap_out, time_out, poses_out,
          pstate_out, sframe_out, i_out, j_out, w_out, v_out, idx_out,
          imap_s, sem_f, sem_i):
    fn = fn_ref[0]
    local = jax.lax.rem(fn, BUFF)
    ts = ts_ref[0]

    both = jnp.dot(wcat_ref[...], xbT_ref[...],
                   preferred_element_type=jnp.float32)
    fmap_out[...] = both[:C]
    imap_s[...] = jnp.tanh(both[C:])
    cp_f = pltpu.make_async_copy(fmap_out, fmap_hbm_out.at[local], sem_f)
    cp_i = pltpu.make_async_copy(imap_s, imap_hbm_out.at[local], sem_i)
    cp_f.start()
    cp_i.start()

    s3 = sc3_ref[...]
    bits3 = jax.lax.bitcast_convert_type(s3 * s3, jnp.int32)

    def bisect(_, carry):
        lo, hi = carry
        mid = lo + jax.lax.div(hi - lo, 2)
        cnt = jnp.sum((bits3 >= mid).astype(jnp.int32))
        big = cnt >= P
        return (jnp.where(big, mid, lo), jnp.where(big, hi, mid))

    t_bits, _ = jax.lax.fori_loop(0, 31, bisect, (jnp.int32(0),
                                                  jnp.int32(0x7F800000)))

    loc_iota = (jax.lax.broadcasted_iota(jnp.int32, (8, 128), 0) * FLS_W
                + jax.lax.broadcasted_iota(jnp.int32, (8, 128), 1))
    flat512 = (jax.lax.broadcasted_iota(jnp.int32, (4, 128), 0) * 128
               + jax.lax.broadcasted_iota(jnp.int32, (4, 128), 1))

    def blk_body(b, carry):
        cv, ci, w = carry
        blk = sc3_ref[b]
        bbits = jax.lax.bitcast_convert_type(blk * blk, jnp.int32)
        mask = (bbits >= t_bits).astype(jnp.int32)
        b_base = jax.lax.div(b, 4) * (FLS_W * 8) + jax.lax.rem(b, 4) * 128

        def cond(c):
            m, _, _, w2 = c
            return jnp.logical_and(jnp.max(m) > 0, w2 < KCAND)

        def step(c):
            m, cv2, ci2, w2 = c
            candm = jnp.where(m > 0, loc_iota, jnp.int32(2 ** 30))
            mpos = jnp.min(candm)
            val = jnp.max(jnp.where(loc_iota == mpos, bbits, jnp.int32(0)))
            sel = flat512 == w2
            cv2 = jnp.where(sel, val, cv2)
            ci2 = jnp.where(sel, b_base + mpos, ci2)
            m = jnp.where(loc_iota == mpos, jnp.int32(0), m)
            return (m, cv2, ci2, w2 + 1)

        _, cv, ci, w = jax.lax.while_loop(cond, step, (mask, cv, ci, w))
        return (cv, ci, w)

    cv0 = jnp.full((4, 128), -1, jnp.int32)
    ci0 = jnp.full((4, 128), 2 ** 30, jnp.int32)
    cand_val, cand_idx, nc = jax.lax.fori_loop(
        0, NBLK, blk_body, (cv0, ci0, jnp.int32(0)))

    sel_iota = (jax.lax.broadcasted_iota(jnp.int32, (2, 128), 0) * 128
                + jax.lax.broadcasted_iota(jnp.int32, (2, 128), 1))

    def sel_body(k, carry):
        msk, sidx = carry
        mskb = msk > 0
        mval = jnp.max(jnp.where(mskb, cand_val, jnp.int32(-2)))
        midx = jnp.min(jnp.where(
            jnp.logical_and(mskb, cand_val == mval), cand_idx,
            jnp.int32(2 ** 30)))
        sidx = jnp.where(sel_iota == k, midx, sidx)
        msk = jnp.where(cand_idx == midx, jnp.int32(0), msk)
        return (msk, sidx)

    mask0 = jnp.ones((4, 128), jnp.int32)
    _, idx_sorted = jax.lax.fori_loop(
        0, P, sel_body, (mask0, jnp.zeros((2, 128), jnp.int32)))
    idx_out[...] = idx_sorted

    ys = jax.lax.div(idx_sorted, FLS_W)
    xs = jax.lax.rem(idx_sorted, FLS_W)
    r = (ys.astype(jnp.float32) / FLS_H) * (R_MAX - R_MIN) + R_MIN
    theta = (xs.astype(jnp.float32) / FLS_W - 0.5) * (
        FOV_H * jnp.pi / 180.0)
    phi = jnp.zeros((2, 128), jnp.float32)
    new_state = jnp.concatenate(
        [r[None], theta[None], phi[None]], axis=0)
    row3 = jax.lax.broadcasted_iota(jnp.int32, (BUFF, 3, 2, 128), 0)
    pstate_out[...] = jnp.where(row3 == local, new_state[None],
                                pstate_ref[...])

    row2 = jax.lax.broadcasted_iota(jnp.int32, (BUFF, P), 0)
    sframe_out[...] = jnp.where(row2 == local, fn, sframe_ref[...])
    lane16 = jax.lax.broadcasted_iota(jnp.int32, (1, BUFF), 1)
    time_out[...] = jnp.where(lane16 == local, ts, time_ref[...])

    k1 = jax.lax.rem(local - 1 + BUFF, BUFF)
    k2 = jax.lax.rem(local - 2 + BUFF, BUFF)
    tvec = time_ref[...]
    l16 = lane16
    t1 = jnp.sum(jnp.where(l16 == k1, tvec, 0.0))
    t2 = jnp.sum(jnp.where(l16 == k2, tvec, 0.0))
    x1 = poses_ref[pl.ds(k1, 1), :]
    x2 = poses_ref[pl.ds(k2, 1), :]
    dt_ratio = (ts - t1) / (t1 - t2)
    new_pose = x1[:, 0:3] + (x1[:, 0:3] - x2[:, 0:3]) * dt_ratio
    q1 = x1[:, 3:7]
    q2 = x2[:, 3:7]
    dot12 = jnp.sum(q1 * q2)
    q1 = jnp.where(dot12 < 0, -q1, q1)
    x1q, y1q, z1q, w1q = q1[:, 0:1], q1[:, 1:2], q1[:, 2:3], q1[:, 3:4]
    x2q, y2q, z2q, w2q = -q2[:, 0:1], -q2[:, 1:2], -q2[:, 2:3], q2[:, 3:4]
    dw = w1q * w2q - x1q * x2q - y1q * y2q - z1q * z2q
    dx = w1q * x2q + x1q * w2q + y1q * z2q - z1q * y2q
    dy = w1q * y2q - x1q * z2q + y1q * w2q + z1q * x2q
    dz = w1q * z2q + x1q * y2q - y1q * x2q + z1q * w2q
    s_ = jnp.sqrt(jnp.clip(1.0 - dw * dw, 0.0, None))
    small = s_ < 1e-3
    denom = jnp.maximum(s_, 1e-12)
    ax = jnp.where(small, 1.0, dx / denom)
    ay = jnp.where(small, 0.0, dy / denom)
    az = jnp.where(small, 0.0, dz / denom)
    dwc = jnp.clip(dw, -1.0, 1.0)
    adw = jnp.abs(dwc)
    acos_pos = jnp.sqrt(jnp.maximum(1.0 - adw, 0.0)) * (
        1.5707288 + adw * (-0.2121144 + adw * (0.0742610 - adw * 0.0187293)))
    acos_dw = jnp.where(dwc < 0, jnp.float32(jnp.pi) - acos_pos, acos_pos)
    rot_angle = 2.0 * acos_dw
    rot_a = rot_angle * dt_ratio
    sh = jnp.sin(rot_a / 2.0)
    ch = jnp.cos(rot_a / 2.0)
    qsx, qsy, qsz, qsw = ax * sh, ay * sh, az * sh, ch
    q0w = qsw * w1q - qsx * x1q - qsy * y1q - qsz * z1q
    q0x = qsw * x1q + qsx * w1q + qsy * z1q - qsz * y1q
    q0y = qsw * y1q - qsx * z1q + qsy * w1q + qsz * x1q
    q0z = qsw * z1q + qsx * y1q - qsy * x1q + qsz * w1q
    qn = jnp.sqrt(q0x * q0x + q0y * q0y + q0z * q0z + q0w * q0w)
    x0 = jnp.concatenate(
        [new_pose, q0x / qn, q0y / qn, q0z / qn, q0w / qn], axis=1)
    row7 = jax.lax.broadcasted_iota(jnp.int32, (BUFF, 7), 0)
    poses_out[...] = jnp.where(row7 == local, x0, poses_ref[...])

    lane = jax.lax.broadcasted_iota(jnp.int32, (1, SLOT), 1)
    first = lane < (T * P)
    i_new = jnp.where(first, fn * P + jax.lax.rem(lane, P),
                      (fn - T) * P + (lane - T * P))
    j_new = jnp.where(first, fn - 1 - jax.lax.div(lane, P), fn)
    rows = jax.lax.broadcasted_iota(jnp.int32, (BUFF, SLOT), 0)
    at_local = rows == local
    i_out[...] = jnp.where(at_local, i_new, i_ref[...])
    j_out[...] = jnp.where(at_local, j_new, j_ref[...])
    w_out[...] = jnp.where(at_local, 0.0, w_ref[...])
    v_out[...] = jnp.where(at_local, jnp.int8(1), v_ref[...])

    cp_f.wait()
    cp_i.wait()


def kernel(frame, time_stamp, frame_n, W_f, W_i, time_buf, poses_buf,
           fmap_buf, imap_buf, patches_buf, patch_state, source_frame,
           i_buf, j_buf, w_buf, v_buf):
    fn1 = jnp.asarray(frame_n, jnp.int32).reshape(1)
    x = frame[0, 0]
    xbT = x.reshape(FH, DOWN, FW, DOWN).transpose(1, 3, 0, 2).reshape(
        DOWN * DOWN, FH * FW)
    sc3 = x.reshape(FLS_H // 8, 8, 4, 128).transpose(0, 2, 1, 3).reshape(
        NBLK, 8, 128)
    wcat = jnp.concatenate([W_f.T, W_i.T], axis=0)

    fmap_hbm = fmap_buf.reshape(BUFF, C, FH * FW)
    imap_hbm = imap_buf.reshape(BUFF, C, FH * FW)
    pstate_t = patch_state.transpose(0, 2, 1).reshape(BUFF, 3, 2, 128)
    time2 = time_buf.reshape(1, BUFF)
    i2 = i_buf.reshape(BUFF, SLOT)
    j2 = j_buf.reshape(BUFF, SLOT)
    w2 = w_buf.reshape(BUFF, SLOT)
    v2 = v_buf.reshape(BUFF, SLOT).astype(jnp.int8)

    vm = pltpu.MemorySpace.VMEM
    hb = pltpu.MemorySpace.HBM
    sm = pltpu.MemorySpace.SMEM
    outs = pl.pallas_call(
        _body,
        in_specs=[
            pl.BlockSpec(memory_space=vm),
            pl.BlockSpec(memory_space=vm),
            pl.BlockSpec(memory_space=vm),
            pl.BlockSpec(memory_space=sm),
            pl.BlockSpec(memory_space=sm),
            pl.BlockSpec(memory_space=vm),
            pl.BlockSpec(memory_space=vm),
            pl.BlockSpec(memory_space=vm),
            pl.BlockSpec(memory_space=vm),
            pl.BlockSpec(memory_space=vm),
            pl.BlockSpec(memory_space=vm),
            pl.BlockSpec(memory_space=vm),
            pl.BlockSpec(memory_space=vm),
            pl.BlockSpec(memory_space=hb),
            pl.BlockSpec(memory_space=hb),
        ],
        out_specs=[
            pl.BlockSpec(memory_space=hb),
            pl.BlockSpec(memory_space=hb),
            pl.BlockSpec(memory_space=vm),
            pl.BlockSpec(memory_space=vm),
            pl.BlockSpec(memory_space=vm),
            pl.BlockSpec(memory_space=vm),
            pl.BlockSpec(memory_space=vm),
            pl.BlockSpec(memory_space=vm),
            pl.BlockSpec(memory_space=vm),
            pl.BlockSpec(memory_space=vm),
            pl.BlockSpec(memory_space=vm),
            pl.BlockSpec(memory_space=vm),
        ],
        out_shape=[
            jax.ShapeDtypeStruct((BUFF, C, FH * FW), jnp.float32),
            jax.ShapeDtypeStruct((BUFF, C, FH * FW), jnp.float32),
            jax.ShapeDtypeStruct((C, FH * FW), jnp.float32),
            jax.ShapeDtypeStruct((1, BUFF), jnp.float32),
            jax.ShapeDtypeStruct((BUFF, 7), jnp.float32),
            jax.ShapeDtypeStruct((BUFF, 3, 2, 128), jnp.float32),
            jax.ShapeDtypeStruct((BUFF, P), jnp.int32),
            jax.ShapeDtypeStruct((BUFF, SLOT), jnp.int32),
            jax.ShapeDtypeStruct((BUFF, SLOT), jnp.int32),
            jax.ShapeDtypeStruct((BUFF, SLOT), jnp.float32),
            jax.ShapeDtypeStruct((BUFF, SLOT), jnp.int8),
            jax.ShapeDtypeStruct((2, 128), jnp.int32),
        ],
        scratch_shapes=[
            pltpu.VMEM((C, FH * FW), jnp.float32),
            pltpu.SemaphoreType.DMA,
            pltpu.SemaphoreType.DMA,
        ],
        input_output_aliases={13: 0, 14: 1},
    )(xbT, sc3, wcat, fn1, time_stamp, time2, poses_buf, pstate_t,
      source_frame, i2, j2, w2, v2, fmap_hbm, imap_hbm)

    (fmap_o, imap_o, fmap_s, time_o, poses_o, pstate_o, sframe_o,
     i_o, j_o, w_o, v_o, idx_o) = outs

    fn_i = jnp.asarray(frame_n, jnp.int32)
    local = fn_i % BUFF
    idx = idx_o.reshape(P)
    ys = idx // FLS_W
    xs = idx % FLS_W
    cy = jnp.clip(ys // DOWN - PS // 2, 0, FH - PS)
    cx = jnp.clip(xs // DOWN - PS // 2, 0, FW - PS)
    rows = cy[:, None] + jnp.arange(PS)
    cols = cx[:, None] + jnp.arange(PS)
    fmap3 = fmap_s.reshape(C, FH, FW)
    new_patches = fmap3[:, rows[:, :, None], cols[:, None, :]]
    new_patches = new_patches.transpose(1, 0, 2, 3)
    patches_o = patches_buf.at[local].set(new_patches)

    return (fmap_o.reshape(BUFF, C, FH, FW),
            imap_o.reshape(BUFF, C, FH, FW),
            patches_o,
            pstate_o.reshape(BUFF, 3, P).transpose(0, 2, 1),
            poses_o,
            time_o.reshape(BUFF),
            sframe_o,
            i_o.reshape(MAX_EDGES),
            j_o.reshape(MAX_EDGES),
            w_o.reshape(MAX_EDGES),
            (v_o != 0).reshape(MAX_EDGES))


# speedup vs baseline: 1.6067x; 1.6067x over previous
import jax
import jax.numpy as jnp
from jax.experimental import pallas as pl
from jax.experimental.pallas import tpu as pltpu

R_MIN = 0.5
R_MAX = 30.0
FLS_H = 512
FLS_W = 512
FOV_H = 130.0
BUFF = 16
P = 256
PS = 8
T = 8
C = 64
DOWN = 4
FH = FLS_H // DOWN
FW = FLS_W // DOWN
SLOT = 2 * P * T
MAX_EDGES = BUFF * SLOT
NPIX = FLS_H * FLS_W
NBLK = 256
KCAND = 512


def _body(xbT_ref, sc3_ref, wcat_ref, fn_ref, ts_ref, time_ref, poses_ref,
          pstate_ref, sframe_ref, i_ref, j_ref, w_ref, v_ref,
          fmap_hbm_in, imap_hbm_in,
          fmap_hbm_out, imap_hbm_out, fmap_out, time_out, poses_out,
          pstate_out, sframe_out, i_out, j_out, w_out, v_out, idx_out,
          imap_s, sem_f, sem_i):
    fn = fn_ref[0]
    local = jax.lax.rem(fn, BUFF)
    ts = ts_ref[0]

    both = jnp.dot(wcat_ref[...], xbT_ref[...],
                   preferred_element_type=jnp.float32)
    fmap_out[...] = both[:C]
    imap_s[...] = jnp.tanh(both[C:])
    cp_f = pltpu.make_async_copy(fmap_out, fmap_hbm_out.at[local], sem_f)
    cp_i = pltpu.make_async_copy(imap_s, imap_hbm_out.at[local], sem_i)
    cp_f.start()
    cp_i.start()

    s2 = sc3_ref[...]
    bits2 = jax.lax.bitcast_convert_type(s2 * s2, jnp.int32)

    def bisect(_, carry):
        lo, hi = carry
        mid = lo + jax.lax.div(hi - lo, 2)
        cnt = jnp.sum((bits2 >= mid).astype(jnp.int32))
        big = cnt >= P
        return (jnp.where(big, mid, lo), jnp.where(big, hi, mid))

    t_bits, _ = jax.lax.fori_loop(0, 31, bisect, (jnp.int32(0),
                                                  jnp.int32(0x7F800000)))

    riota = jax.lax.broadcasted_iota(jnp.int32, (2048, 128), 0)
    liota = jax.lax.broadcasted_iota(jnp.int32, (2048, 128), 1)
    gidx = riota * 128 + liota
    mask_b = bits2 >= t_bits
    mask_i = mask_b.astype(jnp.int32)
    csum = mask_i
    for d in (1, 2, 4, 8, 16, 32, 64, 128, 256, 512, 1024):
        csum = csum + jnp.where(riota >= d, jnp.roll(csum, d, axis=0), 0)
    rank = csum - mask_i

    vrows = []
    irows = []
    for slot in range(16):
        eqb = jnp.logical_and(mask_b, rank == slot)
        vrows.append(jnp.sum(jnp.where(eqb, bits2, 0), axis=0))
        irows.append(jnp.sum(jnp.where(eqb, gidx, 0), axis=0))
    v16 = jnp.stack(vrows, axis=0)
    i16 = jnp.stack(irows, axis=0)
    si = jax.lax.broadcasted_iota(jnp.int32, (16, 128), 0)
    li = jax.lax.broadcasted_iota(jnp.int32, (16, 128), 1)
    empty = v16 == 0
    nv = jnp.where(empty, jnp.int32(1), -v16)
    ii = jnp.where(empty, (1 << 25) + si * 128 + li, i16)

    def partner(a, d):
        if d < 128:
            lo_side = (li & d) == 0
            return jnp.where(lo_side, jnp.roll(a, -d, axis=1),
                             jnp.roll(a, d, axis=1))
        ds = d // 128
        lo_side = (si & ds) == 0
        return jnp.where(lo_side, jnp.roll(a, -ds, axis=0),
                         jnp.roll(a, ds, axis=0))

    f_iota = si * 128 + li
    for k in range(1, 12):
        for j in reversed(range(k)):
            d = 1 << j
            nv_p = partner(nv, d)
            ii_p = partner(ii, d)
            is_lower = (f_iota & d) == 0
            want_min = (jax.lax.shift_right_logical(f_iota, k) & 1) == 0
            self_le = jnp.logical_or(
                nv < nv_p, jnp.logical_and(nv == nv_p, ii <= ii_p))
            keep_self = (is_lower == want_min) == self_le
            nv = jnp.where(keep_self, nv, nv_p)
            ii = jnp.where(keep_self, ii, ii_p)

    idx_sorted = ii[0:2, :]
    idx_out[...] = idx_sorted

    ys = jax.lax.div(idx_sorted, FLS_W)
    xs = jax.lax.rem(idx_sorted, FLS_W)
    r = (ys.astype(jnp.float32) / FLS_H) * (R_MAX - R_MIN) + R_MIN
    theta = (xs.astype(jnp.float32) / FLS_W - 0.5) * (
        FOV_H * jnp.pi / 180.0)
    phi = jnp.zeros((2, 128), jnp.float32)
    new_state = jnp.concatenate(
        [r[None], theta[None], phi[None]], axis=0)
    row3 = jax.lax.broadcasted_iota(jnp.int32, (BUFF, 3, 2, 128), 0)
    pstate_out[...] = jnp.where(row3 == local, new_state[None],
                                pstate_ref[...])

    row2 = jax.lax.broadcasted_iota(jnp.int32, (BUFF, P), 0)
    sframe_out[...] = jnp.where(row2 == local, fn, sframe_ref[...])
    lane16 = jax.lax.broadcasted_iota(jnp.int32, (1, BUFF), 1)
    time_out[...] = jnp.where(lane16 == local, ts, time_ref[...])

    k1 = jax.lax.rem(local - 1 + BUFF, BUFF)
    k2 = jax.lax.rem(local - 2 + BUFF, BUFF)
    tvec = time_ref[...]
    l16 = lane16
    t1 = jnp.sum(jnp.where(l16 == k1, tvec, 0.0))
    t2 = jnp.sum(jnp.where(l16 == k2, tvec, 0.0))
    x1 = poses_ref[pl.ds(k1, 1), :]
    x2 = poses_ref[pl.ds(k2, 1), :]
    dt_ratio = (ts - t1) / (t1 - t2)
    new_pose = x1[:, 0:3] + (x1[:, 0:3] - x2[:, 0:3]) * dt_ratio
    q1 = x1[:, 3:7]
    q2 = x2[:, 3:7]
    dot12 = jnp.sum(q1 * q2)
    q1 = jnp.where(dot12 < 0, -q1, q1)
    x1q, y1q, z1q, w1q = q1[:, 0:1], q1[:, 1:2], q1[:, 2:3], q1[:, 3:4]
    x2q, y2q, z2q, w2q = -q2[:, 0:1], -q2[:, 1:2], -q2[:, 2:3], q2[:, 3:4]
    dw = w1q * w2q - x1q * x2q - y1q * y2q - z1q * z2q
    dx = w1q * x2q + x1q * w2q + y1q * z2q - z1q * y2q
    dy = w1q * y2q - x1q * z2q + y1q * w2q + z1q * x2q
    dz = w1q * z2q + x1q * y2q - y1q * x2q + z1q * w2q
    s_ = jnp.sqrt(jnp.clip(1.0 - dw * dw, 0.0, None))
    small = s_ < 1e-3
    denom = jnp.maximum(s_, 1e-12)
    ax = jnp.where(small, 1.0, dx / denom)
    ay = jnp.where(small, 0.0, dy / denom)
    az = jnp.where(small, 0.0, dz / denom)
    dwc = jnp.clip(dw, -1.0, 1.0)
    adw = jnp.abs(dwc)
    acos_pos = jnp.sqrt(jnp.maximum(1.0 - adw, 0.0)) * (
        1.5707288 + adw * (-0.2121144 + adw * (0.0742610 - adw * 0.0187293)))
    acos_dw = jnp.where(dwc < 0, jnp.float32(jnp.pi) - acos_pos, acos_pos)
    rot_angle = 2.0 * acos_dw
    rot_a = rot_angle * dt_ratio
    sh = jnp.sin(rot_a / 2.0)
    ch = jnp.cos(rot_a / 2.0)
    qsx, qsy, qsz, qsw = ax * sh, ay * sh, az * sh, ch
    q0w = qsw * w1q - qsx * x1q - qsy * y1q - qsz * z1q
    q0x = qsw * x1q + qsx * w1q + qsy * z1q - qsz * y1q
    q0y = qsw * y1q - qsx * z1q + qsy * w1q + qsz * x1q
    q0z = qsw * z1q + qsx * y1q - qsy * x1q + qsz * w1q
    qn = jnp.sqrt(q0x * q0x + q0y * q0y + q0z * q0z + q0w * q0w)
    x0 = jnp.concatenate(
        [new_pose, q0x / qn, q0y / qn, q0z / qn, q0w / qn], axis=1)
    row7 = jax.lax.broadcasted_iota(jnp.int32, (BUFF, 7), 0)
    poses_out[...] = jnp.where(row7 == local, x0, poses_ref[...])

    lane = jax.lax.broadcasted_iota(jnp.int32, (1, SLOT), 1)
    first = lane < (T * P)
    i_new = jnp.where(first, fn * P + jax.lax.rem(lane, P),
                      (fn - T) * P + (lane - T * P))
    j_new = jnp.where(first, fn - 1 - jax.lax.div(lane, P), fn)
    rows = jax.lax.broadcasted_iota(jnp.int32, (BUFF, SLOT), 0)
    at_local = rows == local
    i_out[...] = jnp.where(at_local, i_new, i_ref[...])
    j_out[...] = jnp.where(at_local, j_new, j_ref[...])
    w_out[...] = jnp.where(at_local, 0.0, w_ref[...])
    v_out[...] = jnp.where(at_local, jnp.int8(1), v_ref[...])

    cp_f.wait()
    cp_i.wait()


def kernel(frame, time_stamp, frame_n, W_f, W_i, time_buf, poses_buf,
           fmap_buf, imap_buf, patches_buf, patch_state, source_frame,
           i_buf, j_buf, w_buf, v_buf):
    fn1 = jnp.asarray(frame_n, jnp.int32).reshape(1)
    x = frame[0, 0]
    xbT = x.reshape(FH, DOWN, FW, DOWN).transpose(1, 3, 0, 2).reshape(
        DOWN * DOWN, FH * FW)
    sc2 = x.reshape(2048, 128)
    wcat = jnp.concatenate([W_f.T, W_i.T], axis=0)

    fmap_hbm = fmap_buf.reshape(BUFF, C, FH * FW)
    imap_hbm = imap_buf.reshape(BUFF, C, FH * FW)
    pstate_t = patch_state.transpose(0, 2, 1).reshape(BUFF, 3, 2, 128)
    time2 = time_buf.reshape(1, BUFF)
    i2 = i_buf.reshape(BUFF, SLOT)
    j2 = j_buf.reshape(BUFF, SLOT)
    w2 = w_buf.reshape(BUFF, SLOT)
    v2 = v_buf.reshape(BUFF, SLOT).astype(jnp.int8)

    vm = pltpu.MemorySpace.VMEM
    hb = pltpu.MemorySpace.HBM
    sm = pltpu.MemorySpace.SMEM
    outs = pl.pallas_call(
        _body,
        in_specs=[
            pl.BlockSpec(memory_space=vm),
            pl.BlockSpec(memory_space=vm),
            pl.BlockSpec(memory_space=vm),
            pl.BlockSpec(memory_space=sm),
            pl.BlockSpec(memory_space=sm),
            pl.BlockSpec(memory_space=vm),
            pl.BlockSpec(memory_space=vm),
            pl.BlockSpec(memory_space=vm),
            pl.BlockSpec(memory_space=vm),
            pl.BlockSpec(memory_space=vm),
            pl.BlockSpec(memory_space=vm),
            pl.BlockSpec(memory_space=vm),
            pl.BlockSpec(memory_space=vm),
            pl.BlockSpec(memory_space=hb),
            pl.BlockSpec(memory_space=hb),
        ],
        out_specs=[
            pl.BlockSpec(memory_space=hb),
            pl.BlockSpec(memory_space=hb),
            pl.BlockSpec(memory_space=vm),
            pl.BlockSpec(memory_space=vm),
            pl.BlockSpec(memory_space=vm),
            pl.BlockSpec(memory_space=vm),
            pl.BlockSpec(memory_space=vm),
            pl.BlockSpec(memory_space=vm),
            pl.BlockSpec(memory_space=vm),
            pl.BlockSpec(memory_space=vm),
            pl.BlockSpec(memory_space=vm),
            pl.BlockSpec(memory_space=vm),
        ],
        out_shape=[
            jax.ShapeDtypeStruct((BUFF, C, FH * FW), jnp.float32),
            jax.ShapeDtypeStruct((BUFF, C, FH * FW), jnp.float32),
            jax.ShapeDtypeStruct((C, FH * FW), jnp.float32),
            jax.ShapeDtypeStruct((1, BUFF), jnp.float32),
            jax.ShapeDtypeStruct((BUFF, 7), jnp.float32),
            jax.ShapeDtypeStruct((BUFF, 3, 2, 128), jnp.float32),
            jax.ShapeDtypeStruct((BUFF, P), jnp.int32),
            jax.ShapeDtypeStruct((BUFF, SLOT), jnp.int32),
            jax.ShapeDtypeStruct((BUFF, SLOT), jnp.int32),
            jax.ShapeDtypeStruct((BUFF, SLOT), jnp.float32),
            jax.ShapeDtypeStruct((BUFF, SLOT), jnp.int8),
            jax.ShapeDtypeStruct((2, 128), jnp.int32),
        ],
        scratch_shapes=[
            pltpu.VMEM((C, FH * FW), jnp.float32),
            pltpu.SemaphoreType.DMA,
            pltpu.SemaphoreType.DMA,
        ],
        input_output_aliases={13: 0, 14: 1},
    )(xbT, sc2, wcat, fn1, time_stamp, time2, poses_buf, pstate_t,
      source_frame, i2, j2, w2, v2, fmap_hbm, imap_hbm)

    (fmap_o, imap_o, fmap_s, time_o, poses_o, pstate_o, sframe_o,
     i_o, j_o, w_o, v_o, idx_o) = outs

    fn_i = jnp.asarray(frame_n, jnp.int32)
    local = fn_i % BUFF
    idx = idx_o.reshape(P)
    ys = idx // FLS_W
    xs = idx % FLS_W
    cy = jnp.clip(ys // DOWN - PS // 2, 0, FH - PS)
    cx = jnp.clip(xs // DOWN - PS // 2, 0, FW - PS)
    rows = cy[:, None] + jnp.arange(PS)
    cols = cx[:, None] + jnp.arange(PS)
    fmap3 = fmap_s.reshape(C, FH, FW)
    new_patches = fmap3[:, rows[:, :, None], cols[:, None, :]]
    new_patches = new_patches.transpose(1, 0, 2, 3)
    patches_o = patches_buf.at[local].set(new_patches)

    return (fmap_o.reshape(BUFF, C, FH, FW),
            imap_o.reshape(BUFF, C, FH, FW),
            patches_o,
            pstate_o.reshape(BUFF, 3, P).transpose(0, 2, 1),
            poses_o,
            time_o.reshape(BUFF),
            sframe_o,
            i_o.reshape(MAX_EDGES),
            j_o.reshape(MAX_EDGES),
            w_o.reshape(MAX_EDGES),
            (v_o != 0).reshape(MAX_EDGES))
